# baseline (speedup 1.0000x reference)
import functools

import jax
import jax.numpy as jnp
from jax import lax
from jax.experimental import pallas as pl
from jax.experimental.pallas import tpu as pltpu
from jax.experimental.pallas import tpu_sc as plsc

N = 100000
A = 3200000

NC = 2
NS = 16
NW = NC * NS
L = 16

T = A // NW
B = 400
NBLK = T // B
NJ = B // L

_MAGIC = 0x5F3759DF

_ACOS = (1.5707963050, -0.2145988016, 0.0889789874, -0.0501743046,
         0.0308918810, -0.0170881256, 0.0066700901, -0.0012624911)


def _rsqrt(x, iters=3):
    i = plsc.bitcast(x, jnp.int32)
    y = plsc.bitcast(jnp.int32(_MAGIC) - (i >> 1), jnp.float32)
    for _ in range(iters):
        y = y * (1.5 - 0.5 * x * y * y)
    return y


def _acos(x):
    t = jnp.abs(x)
    u = 1.0 - t
    s = u * _rsqrt(jnp.maximum(u, 1e-30))
    p = jnp.full((L,), _ACOS[7], dtype=jnp.float32)
    for c in _ACOS[6::-1]:
        p = p * t + c
    r = s * p
    return jnp.where(x < 0, jnp.float32(jnp.pi) - r, r)


def _sc_body(tab_hbm, ang_hbm, th_hbm, kk_hbm, out_hbm,
             tab_sh, ang_v, idx_v, rows_v, thk_v, acc_v, semS, semG, semT):
    cid = lax.axis_index("c")
    sid = lax.axis_index("s")
    wid = sid * NC + cid

    @pl.when(sid == 0)
    def _():
        pltpu.sync_copy(tab_hbm, tab_sh)
    plsc.subcore_barrier()

    base = wid * T
    iota = lax.iota(jnp.int32, L)
    cvec = [jnp.full((L,), c, jnp.int32) for c in range(3)]

    def fire_S(b, par):
        off = base + b * B
        pltpu.async_copy(ang_hbm.at[pl.ds(off, B)], ang_v[par], semS[par])

    def wait_S(par):
        pltpu.make_async_copy(ang_hbm.at[pl.ds(0, B)], ang_v[par],
                              semS[par]).wait()

    def extract(par):
        av = ang_v[par]
        i0, ic, i2 = idx_v[par]

        def e_body(j, carry):
            rid = j * L + iota
            sl = pl.ds(j * L, L)
            i0[sl] = plsc.load_gather(av, [rid, cvec[0]])
            ic[sl] = plsc.load_gather(av, [rid, cvec[1]])
            i2[sl] = plsc.load_gather(av, [rid, cvec[2]])
            return carry

        lax.fori_loop(0, NJ, e_body, 0)

    def fire_G(b, par):
        off = base + b * B
        for e in range(3):
            pltpu.async_copy(tab_sh.at[idx_v[par][e]], rows_v[par][e],
                             semG[par])
        pltpu.async_copy(th_hbm.at[pl.ds(off, B)], thk_v[par][0], semT[par])
        pltpu.async_copy(kk_hbm.at[pl.ds(off, B)], thk_v[par][1], semT[par])

    def wait_G(par):
        for e in range(3):
            pltpu.make_async_copy(tab_sh.at[idx_v[par][e]], rows_v[par][e],
                                  semG[par]).wait()
        for w in range(2):
            pltpu.make_async_copy(th_hbm.at[pl.ds(0, B)], thk_v[par][w],
                                  semT[par]).wait()

    def compute(par, acc):
        r0, rc, r2 = rows_v[par]
        th_ref, kk_ref = thk_v[par]

        def j_body(j, acc):
            rid = j * L + iota
            p0 = [plsc.load_gather(r0, [rid, cvec[c]]) for c in range(3)]
            pc = [plsc.load_gather(rc, [rid, cvec[c]]) for c in range(3)]
            p2 = [plsc.load_gather(r2, [rid, cvec[c]]) for c in range(3)]
            v1x, v1y, v1z = (p0[0] - pc[0], p0[1] - pc[1], p0[2] - pc[2])
            v2x, v2y, v2z = (p2[0] - pc[0], p2[1] - pc[1], p2[2] - pc[2])
            dot = v1x * v2x + v1y * v2y + v1z * v2z
            n1 = v1x * v1x + v1y * v1y + v1z * v1z
            n2 = v2x * v2x + v2y * v2y + v2z * v2z
            inv = _rsqrt(jnp.maximum(n1 * n2, 1e-30))
            cos = jnp.clip(dot * inv, -1.0, 1.0)
            theta = _acos(cos)
            sl = pl.ds(j * L, L)
            dth = theta - th_ref[sl]
            return acc + dth * dth * kk_ref[sl] * 0.5

        return lax.fori_loop(0, NJ, j_body, acc)

    fire_S(0, 0)
    fire_S(1, 1)
    wait_S(0)
    extract(0)
    fire_G(0, 0)

    def pair_body(p, acc):
        b = 2 * p
        wait_S(1)
        extract(1)
        fire_G(b + 1, 1)
        wait_G(0)

        @pl.when(b + 2 < NBLK)
        def _():
            fire_S(b + 2, 0)
        acc = compute(0, acc)

        @pl.when(b + 2 < NBLK)
        def _():
            wait_S(0)
            extract(0)
            fire_G(b + 2, 0)
        wait_G(1)

        @pl.when(b + 3 < NBLK)
        def _():
            fire_S(b + 3, 1)
        acc = compute(1, acc)
        return acc

    acc = lax.fori_loop(0, NBLK // 2, pair_body, jnp.zeros((L,), jnp.float32))
    acc_v[...] = acc
    pltpu.sync_copy(acc_v, out_hbm.at[wid])


@jax.jit
def _sc_call(tab, ang, theta0, k):
    mesh = plsc.VectorSubcoreMesh(core_axis_name="c", subcore_axis_name="s")
    f = functools.partial(
        pl.kernel,
        out_type=jax.ShapeDtypeStruct((NW, L), jnp.float32),
        mesh=mesh,
        scratch_types=(
            [pltpu.VMEM_SHARED((N, 8), jnp.float32)]
            + [[pltpu.VMEM((B, 3), jnp.int32)] * 2]
            + [[[pltpu.VMEM((B,), jnp.int32)] * 3] * 2]
            + [[[pltpu.VMEM((B, 8), jnp.float32)] * 3] * 2]
            + [[[pltpu.VMEM((B,), jnp.float32)] * 2] * 2]
            + [pltpu.VMEM((L,), jnp.float32)]
            + [[pltpu.SemaphoreType.DMA] * 2] * 3
        ),
        compiler_params=pltpu.CompilerParams(
            needs_layout_passes=False, use_tc_tiling_on_sc=False),
    )(_sc_body)
    return f(tab, ang, theta0, k)


def kernel(coords, angles, theta0, k):
    tab = jnp.concatenate([coords, jnp.zeros((N, 5), jnp.float32)], axis=1)
    partials = _sc_call(tab, angles, theta0, k)
    return jnp.sum(partials)


# speedup vs baseline: 36.6447x; 36.6447x over previous
import functools

import jax
import jax.numpy as jnp
from jax import lax
from jax.experimental import pallas as pl
from jax.experimental.pallas import tpu as pltpu
from jax.experimental.pallas import tpu_sc as plsc

N = 100000
A = 3200000

NC = 2
NS = 16
NW = NC * NS
L = 16

T = A // NW
B = 800
NBLK = T // B
NJ = B // L

_MAGIC = 0x5F3759DF

_ACOS = (1.5707963050, -0.2145988016, 0.0889789874, -0.0501743046,
         0.0308918810, -0.0170881256, 0.0066700901, -0.0012624911)


def _rsqrt(x, iters=2):
    i = plsc.bitcast(x, jnp.int32)
    y = plsc.bitcast(jnp.int32(_MAGIC) - (i >> 1), jnp.float32)
    for _ in range(iters):
        y = y * (1.5 - 0.5 * x * y * y)
    return y


def _acos(x):
    t = jnp.abs(x)
    u = 1.0 - t
    s = u * _rsqrt(jnp.maximum(u, 1e-30))
    p = jnp.full((L,), _ACOS[7], dtype=jnp.float32)
    for c in _ACOS[6::-1]:
        p = p * t + c
    r = s * p
    return jnp.where(x < 0, jnp.float32(jnp.pi) - r, r)


def _sc_body(tab_hbm, a0_hbm, ac_hbm, a2_hbm, th_hbm, kk_hbm, out_hbm,
             tab_sh, idx_v, rows_v, thk_v, acc_v, semS, semG, semT):
    cid = lax.axis_index("c")
    sid = lax.axis_index("s")
    wid = sid * NC + cid

    @pl.when(sid == 0)
    def _():
        pltpu.sync_copy(tab_hbm, tab_sh)
    plsc.subcore_barrier()

    base = wid * T
    iota = lax.iota(jnp.int32, L)
    idx_hbms = (a0_hbm, ac_hbm, a2_hbm)

    def fire_S(b, par):
        off = base + b * B
        for e in range(3):
            pltpu.async_copy(idx_hbms[e].at[pl.ds(off, B)],
                             idx_v[par][e], semS[par])

    def wait_S(par):
        for e in range(3):
            pltpu.make_async_copy(idx_hbms[e].at[pl.ds(0, B)],
                                  idx_v[par][e], semS[par]).wait()

    def fire_G(b, par):
        off = base + b * B
        for e in range(3):
            pltpu.async_copy(tab_sh.at[idx_v[par][e]], rows_v[par][e],
                             semG[par])
        pltpu.async_copy(th_hbm.at[pl.ds(off, B)], thk_v[par][0], semT[par])
        pltpu.async_copy(kk_hbm.at[pl.ds(off, B)], thk_v[par][1], semT[par])

    def wait_G(par):
        for e in range(3):
            pltpu.make_async_copy(tab_sh.at[idx_v[par][e]], rows_v[par][e],
                                  semG[par]).wait()
        for w in range(2):
            pltpu.make_async_copy(th_hbm.at[pl.ds(0, B)], thk_v[par][w],
                                  semT[par]).wait()

    def compute(par, acc):
        r0, rc, r2 = rows_v[par]
        th_ref, kk_ref = thk_v[par]
        cvec = [jnp.full((L,), c, jnp.int32) for c in range(3)]

        def j_body(j, acc):
            rid = j * L + iota
            p0 = [plsc.load_gather(r0, [rid, cvec[c]]) for c in range(3)]
            pc = [plsc.load_gather(rc, [rid, cvec[c]]) for c in range(3)]
            p2 = [plsc.load_gather(r2, [rid, cvec[c]]) for c in range(3)]
            v1x, v1y, v1z = (p0[0] - pc[0], p0[1] - pc[1], p0[2] - pc[2])
            v2x, v2y, v2z = (p2[0] - pc[0], p2[1] - pc[1], p2[2] - pc[2])
            dot = v1x * v2x + v1y * v2y + v1z * v2z
            n1 = v1x * v1x + v1y * v1y + v1z * v1z
            n2 = v2x * v2x + v2y * v2y + v2z * v2z
            inv = _rsqrt(jnp.maximum(n1 * n2, 1e-30))
            cos = jnp.clip(dot * inv, -1.0, 1.0)
            theta = _acos(cos)
            sl = pl.ds(j * L, L)
            dth = theta - th_ref[sl]
            return acc + dth * dth * kk_ref[sl] * 0.5

        return lax.fori_loop(0, NJ, j_body, acc)

    fire_S(0, 0)
    fire_S(1, 1)
    wait_S(0)
    fire_G(0, 0)

    def pair_body(p, acc):
        b = 2 * p
        wait_S(1)
        fire_G(b + 1, 1)
        wait_G(0)

        @pl.when(b + 2 < NBLK)
        def _():
            fire_S(b + 2, 0)
        acc = compute(0, acc)

        @pl.when(b + 2 < NBLK)
        def _():
            wait_S(0)
            fire_G(b + 2, 0)
        wait_G(1)

        @pl.when(b + 3 < NBLK)
        def _():
            fire_S(b + 3, 1)
        acc = compute(1, acc)
        return acc

    acc = lax.fori_loop(0, NBLK // 2, pair_body, jnp.zeros((L,), jnp.float32))
    if NBLK % 2:
        wait_G(0)
        acc = compute(0, acc)
    acc_v[...] = acc
    pltpu.sync_copy(acc_v, out_hbm.at[wid])


@jax.jit
def _sc_call(tab, a0, ac, a2, theta0, k):
    mesh = plsc.VectorSubcoreMesh(core_axis_name="c", subcore_axis_name="s")
    f = functools.partial(
        pl.kernel,
        out_type=jax.ShapeDtypeStruct((NW, L), jnp.float32),
        mesh=mesh,
        scratch_types=(
            [pltpu.VMEM_SHARED((N, 8), jnp.float32)]
            + [[[pltpu.VMEM((B,), jnp.int32)] * 3] * 2]
            + [[[pltpu.VMEM((B, 8), jnp.float32)] * 3] * 2]
            + [[[pltpu.VMEM((B,), jnp.float32)] * 2] * 2]
            + [pltpu.VMEM((L,), jnp.float32)]
            + [[pltpu.SemaphoreType.DMA] * 2] * 3
        ),
        compiler_params=pltpu.CompilerParams(
            needs_layout_passes=False, use_tc_tiling_on_sc=False),
    )(_sc_body)
    return f(tab, a0, ac, a2, theta0, k)


def kernel(coords, angles, theta0, k):
    tab = jnp.concatenate([coords, jnp.zeros((N, 5), jnp.float32)], axis=1)
    a0 = angles[:, 0]
    ac = angles[:, 1]
    a2 = angles[:, 2]
    partials = _sc_call(tab, a0, ac, a2, theta0, k)
    return jnp.sum(partials)


# speedup vs baseline: 37.7329x; 1.0297x over previous
import functools

import jax
import jax.numpy as jnp
from jax import lax
from jax.experimental import pallas as pl
from jax.experimental.pallas import tpu as pltpu
from jax.experimental.pallas import tpu_sc as plsc

N = 100000
A = 3200000

NC = 2
NS = 16
NW = NC * NS
L = 16

T = A // NW
B = 800
NBLK = T // B
NJ = B // L
UNROLL = 2

_MAGIC = 0x5F3759DF

_ACOS = (1.5707288, -0.2121144, 0.0742610, -0.0187293)


def _rsqrt(x, iters=2):
    i = plsc.bitcast(x, jnp.int32)
    y = plsc.bitcast(jnp.int32(_MAGIC) - (i >> 1), jnp.float32)
    for _ in range(iters):
        y = y * (1.5 - 0.5 * x * y * y)
    return y


def _acos(x):
    t = jnp.abs(x)
    u = 1.0 - t
    s = u * _rsqrt(jnp.maximum(u, 1e-30))
    p = jnp.full((L,), _ACOS[3], dtype=jnp.float32)
    for c in _ACOS[2::-1]:
        p = p * t + c
    r = s * p
    return jnp.where(x < 0, jnp.float32(jnp.pi) - r, r)


def _sc_body(tab_hbm, a0_hbm, ac_hbm, a2_hbm, th_hbm, kk_hbm, out_hbm,
             tab_sh, idx_v, rows_v, thk_v, acc_v, semS, semG, semT):
    cid = lax.axis_index("c")
    sid = lax.axis_index("s")
    wid = sid * NC + cid

    @pl.when(sid == 0)
    def _():
        pltpu.sync_copy(tab_hbm, tab_sh)
    plsc.subcore_barrier()

    base = wid * T
    iota = lax.iota(jnp.int32, L)
    idx_hbms = (a0_hbm, ac_hbm, a2_hbm)

    def fire_S(b, par):
        off = base + b * B
        for e in range(3):
            pltpu.async_copy(idx_hbms[e].at[pl.ds(off, B)],
                             idx_v[par].at[pl.ds(e * B, B)], semS[par])

    def wait_S(par):
        for e in range(3):
            pltpu.make_async_copy(idx_hbms[e].at[pl.ds(0, B)],
                                  idx_v[par].at[pl.ds(e * B, B)],
                                  semS[par]).wait()

    def fire_G(b, par):
        off = base + b * B
        pltpu.async_copy(tab_sh.at[idx_v[par]], rows_v[par], semG[par])
        pltpu.async_copy(th_hbm.at[pl.ds(off, B)], thk_v[par][0], semT[par])
        pltpu.async_copy(kk_hbm.at[pl.ds(off, B)], thk_v[par][1], semT[par])

    def wait_G(par):
        pltpu.make_async_copy(tab_sh.at[idx_v[par]], rows_v[par],
                              semG[par]).wait()
        for w in range(2):
            pltpu.make_async_copy(th_hbm.at[pl.ds(0, B)], thk_v[par][w],
                                  semT[par]).wait()

    def compute(par, acc):
        rows = rows_v[par]
        th_ref, kk_ref = thk_v[par]
        cvec = [jnp.full((L,), c, jnp.int32) for c in range(3)]

        def one_chunk(rid, sl):
            p0 = [plsc.load_gather(rows, [rid, cvec[c]]) for c in range(3)]
            pc = [plsc.load_gather(rows, [rid + B, cvec[c]])
                  for c in range(3)]
            p2 = [plsc.load_gather(rows, [rid + 2 * B, cvec[c]])
                  for c in range(3)]
            v1x, v1y, v1z = (p0[0] - pc[0], p0[1] - pc[1], p0[2] - pc[2])
            v2x, v2y, v2z = (p2[0] - pc[0], p2[1] - pc[1], p2[2] - pc[2])
            dot = v1x * v2x + v1y * v2y + v1z * v2z
            n1 = v1x * v1x + v1y * v1y + v1z * v1z
            n2 = v2x * v2x + v2y * v2y + v2z * v2z
            inv = _rsqrt(jnp.maximum(n1 * n2, 1e-30))
            cos = jnp.clip(dot * inv, -1.0, 1.0)
            theta = _acos(cos)
            dth = theta - th_ref[sl]
            return dth * dth * kk_ref[sl] * 0.5

        def j_body(j, acc):
            for q in range(UNROLL):
                jj = j * UNROLL + q
                acc = acc + one_chunk(jj * L + iota, pl.ds(jj * L, L))
            return acc

        return lax.fori_loop(0, NJ // UNROLL, j_body, acc)

    fire_S(0, 0)
    fire_S(1, 1)
    wait_S(0)
    fire_G(0, 0)

    def pair_body(p, acc):
        b = 2 * p
        wait_S(1)
        fire_G(b + 1, 1)
        wait_G(0)

        @pl.when(b + 2 < NBLK)
        def _():
            fire_S(b + 2, 0)
        acc = compute(0, acc)

        @pl.when(b + 2 < NBLK)
        def _():
            wait_S(0)
            fire_G(b + 2, 0)
        wait_G(1)

        @pl.when(b + 3 < NBLK)
        def _():
            fire_S(b + 3, 1)
        acc = compute(1, acc)
        return acc

    acc = lax.fori_loop(0, NBLK // 2, pair_body, jnp.zeros((L,), jnp.float32))
    if NBLK % 2:
        wait_G(0)
        acc = compute(0, acc)
    acc_v[...] = acc
    pltpu.sync_copy(acc_v, out_hbm.at[wid])


@jax.jit
def _sc_call(tab, a0, ac, a2, theta0, k):
    mesh = plsc.VectorSubcoreMesh(core_axis_name="c", subcore_axis_name="s")
    f = functools.partial(
        pl.kernel,
        out_type=jax.ShapeDtypeStruct((NW, L), jnp.float32),
        mesh=mesh,
        scratch_types=(
            [pltpu.VMEM_SHARED((N, 8), jnp.float32)]
            + [[pltpu.VMEM((3 * B,), jnp.int32)] * 2]
            + [[pltpu.VMEM((3 * B, 8), jnp.float32)] * 2]
            + [[[pltpu.VMEM((B,), jnp.float32)] * 2] * 2]
            + [pltpu.VMEM((L,), jnp.float32)]
            + [[pltpu.SemaphoreType.DMA] * 2] * 3
        ),
        compiler_params=pltpu.CompilerParams(
            needs_layout_passes=False, use_tc_tiling_on_sc=False),
    )(_sc_body)
    return f(tab, a0, ac, a2, theta0, k)


def kernel(coords, angles, theta0, k):
    tab = jnp.concatenate([coords, jnp.zeros((N, 5), jnp.float32)], axis=1)
    a0 = angles[:, 0]
    ac = angles[:, 1]
    a2 = angles[:, 2]
    partials = _sc_call(tab, a0, ac, a2, theta0, k)
    return jnp.sum(partials)
